# baseline (speedup 1.0000x reference)
import functools

import jax
import jax.numpy as jnp
from jax import lax
from jax.experimental import pallas as pl
from jax.experimental.pallas import tpu as pltpu
from jax.experimental.pallas import tpu_sc as plsc

B = 16384
D = 64
NF = 128

NC, NS = 2, 16
NW = NC * NS
BPW = B // NW

BB = 1024
NB = B // BB


def _make_sc_gather():
    mesh = plsc.VectorSubcoreMesh(core_axis_name="c", subcore_axis_name="s")

    @functools.partial(
        pl.kernel,
        mesh=mesh,
        compiler_params=pltpu.CompilerParams(use_tc_tiling_on_sc=False),
        out_type=jax.ShapeDtypeStruct((B, D), jnp.float32),
        scratch_types=[
            pltpu.VMEM((BPW,), jnp.int32),
            pltpu.VMEM((BPW, D), jnp.float32),
            pltpu.SemaphoreType.DMA,
        ],
    )
    def gather_k(table_hbm, idx_hbm, out_hbm, idx_v, rows_v, sem):
        wid = lax.axis_index("s") * NC + lax.axis_index("c")
        base = wid * BPW
        pltpu.sync_copy(idx_hbm.at[pl.ds(base, BPW)], idx_v)
        pltpu.async_copy(table_hbm.at[idx_v], rows_v, sem).wait()
        pltpu.sync_copy(rows_v, out_hbm.at[pl.ds(base, BPW)])

    return gather_k


_sc_gather_cache = []


def _sc_gather(table, idx):
    if not _sc_gather_cache:
        _sc_gather_cache.append(_make_sc_gather())
    return _sc_gather_cache[0](table, idx)


def _dense_body(ue_ref, mf_ref, wf_ref, bf_ref, w1_ref, w2_ref, bfc_ref, out_ref):
    me = jnp.dot(mf_ref[...], wf_ref[...],
                 preferred_element_type=jnp.float32) + bf_ref[...]
    r = (jnp.sum(ue_ref[...] * w1_ref[...], axis=1)
         + jnp.sum(me * w2_ref[...], axis=1)
         + bfc_ref[0, 0])
    out_ref[...] = r.reshape(BB // 128, 128)


_dense = pl.pallas_call(
    _dense_body,
    grid=(NB,),
    in_specs=[
        pl.BlockSpec((BB, D), lambda i: (i, 0)),
        pl.BlockSpec((BB, NF), lambda i: (i, 0)),
        pl.BlockSpec((NF, D), lambda i: (0, 0)),
        pl.BlockSpec((1, D), lambda i: (0, 0)),
        pl.BlockSpec((1, D), lambda i: (0, 0)),
        pl.BlockSpec((1, D), lambda i: (0, 0)),
        pl.BlockSpec((1, 1), lambda i: (0, 0)),
    ],
    out_specs=pl.BlockSpec((BB // 128, 128), lambda i: (i, 0)),
    out_shape=jax.ShapeDtypeStruct((B // 128, 128), jnp.float32),
)


def kernel(user_ids, movie_features, user_embedding, W_feat, b_feat, W_fc, b_fc):
    ue = _sc_gather(user_embedding, user_ids.astype(jnp.int32))
    w1 = W_fc[:D, 0].reshape(1, D)
    w2 = W_fc[D:, 0].reshape(1, D)
    out2 = _dense(ue, movie_features, W_feat,
                  b_feat.reshape(1, D), w1, w2, b_fc.reshape(1, 1))
    return out2.reshape(B)


# speedup vs baseline: 1.7250x; 1.7250x over previous
import functools

import jax
import jax.numpy as jnp
from jax import lax
from jax.experimental import pallas as pl
from jax.experimental.pallas import tpu as pltpu
from jax.experimental.pallas import tpu_sc as plsc

B = 16384
D = 64
NF = 128
NC, NS = 2, 16
NW = NC * NS
BPW = B // NW
K = 64
NP = 208

_DNUMS = lax.GatherDimensionNumbers(
    offset_dims=(), collapsed_slice_dims=(0,), start_index_map=(0,))


def _shuf(v, idx):
    return lax.gather(v, idx[:, None], _DNUMS, (1,),
                      mode=lax.GatherScatterMode.PROMISE_IN_BOUNDS)


def _tree_sum(v, lane):
    for d in (1, 2, 4, 8):
        v = v + _shuf(v, lane ^ d)
    return v


def _make_fused():
    mesh = plsc.VectorSubcoreMesh(core_axis_name="c", subcore_axis_name="s")

    @functools.partial(
        pl.kernel,
        mesh=mesh,
        out_type=jax.ShapeDtypeStruct((B,), jnp.float32),
        scratch_types=[
            pltpu.VMEM((BPW,), jnp.int32),
            pltpu.VMEM((BPW, D), jnp.float32),
            pltpu.VMEM((BPW // 4, NF), jnp.float32),
            pltpu.VMEM((BPW // 4, NF), jnp.float32),
            pltpu.VMEM((NF, D), jnp.float32),
            pltpu.VMEM((NP,), jnp.float32),
            pltpu.VMEM((NF,), jnp.float32),
            pltpu.VMEM((BPW,), jnp.float32),
            pltpu.SemaphoreType.DMA,
            pltpu.SemaphoreType.DMA,
            pltpu.SemaphoreType.DMA,
        ],
    )
    def fused_k(table_hbm, idx_hbm, mf_hbm, wfeat_hbm, params_hbm, out_hbm,
                idx_v, rows_v, mf_v0, mf_v1, wfeat_v, par_v, wm_v, out_v,
                sem_g, sem_m0, sem_m1):
        wid = lax.axis_index("s") * NC + lax.axis_index("c")
        base = wid * BPW
        lane = lax.iota(jnp.int32, 16)
        QR = BPW // 4
        mf_bufs = (mf_v0, mf_v1)
        mf_sems = (sem_m0, sem_m1)

        pltpu.sync_copy(idx_hbm.at[pl.ds(base, BPW)], idx_v)
        pltpu.async_copy(mf_hbm.at[pl.ds(base, QR)], mf_v0, sem_m0)
        pltpu.sync_copy(wfeat_hbm, wfeat_v)
        pltpu.sync_copy(params_hbm, par_v)

        KG = K // 16

        def fire(g, carry):
            vec = idx_v[pl.ds(g * 16, 16)]
            for j in range(16):
                pltpu.async_copy(table_hbm.at[pl.ds(vec[j], 1)],
                                 rows_v.at[pl.ds(g * 16 + j, 1)], sem_g)

            @pl.when(g >= KG)
            def _():
                pltpu.make_async_copy(
                    table_hbm.at[pl.ds(0, 16)],
                    rows_v.at[pl.ds((g - KG) * 16, 16)],
                    sem_g).wait()

            return carry

        lax.fori_loop(0, BPW // 16, fire, 0)

        w2c = [par_v[pl.ds(D + 16 * t, 16)] for t in range(4)]

        def wm_body(g, carry):
            ovec = jnp.zeros((16,), jnp.float32)
            for j in range(16):
                r = g * 16 + j
                a = (wfeat_v[r, pl.ds(0, 16)] * w2c[0]
                     + wfeat_v[r, pl.ds(16, 16)] * w2c[1]
                     + wfeat_v[r, pl.ds(32, 16)] * w2c[2]
                     + wfeat_v[r, pl.ds(48, 16)] * w2c[3])
                ovec = jnp.where(lane == j, _tree_sum(a, lane), ovec)
            wm_v[pl.ds(g * 16, 16)] = ovec
            return carry

        lax.fori_loop(0, NF // 16, wm_body, 0)

        ca = (par_v[pl.ds(2 * D, 16)] * w2c[0]
              + par_v[pl.ds(2 * D + 16, 16)] * w2c[1]
              + par_v[pl.ds(2 * D + 32, 16)] * w2c[2]
              + par_v[pl.ds(2 * D + 48, 16)] * w2c[3])
        cvec = _tree_sum(ca, lane) + par_v[pl.ds(3 * D, 16)]

        def drain(g, carry):
            pltpu.make_async_copy(table_hbm.at[pl.ds(0, 16)],
                                  rows_v.at[pl.ds(BPW - K + g * 16, 16)],
                                  sem_g).wait()
            return carry

        lax.fori_loop(0, KG, drain, 0)

        w1c = [par_v[pl.ds(16 * t, 16)] for t in range(4)]
        wmc = [wm_v[pl.ds(16 * t, 16)] for t in range(8)]

        for q in range(4):
            mbuf = mf_bufs[q % 2]
            pltpu.make_async_copy(mf_hbm.at[pl.ds(base + q * QR, QR)],
                                  mbuf, mf_sems[q % 2]).wait()
            if q + 1 < 4:
                pltpu.async_copy(mf_hbm.at[pl.ds(base + (q + 1) * QR, QR)],
                                 mf_bufs[(q + 1) % 2], mf_sems[(q + 1) % 2])

            def body(g, carry, q=q, mbuf=mbuf):
                ovec = cvec
                for j in range(16):
                    r = q * QR + g * 16 + j
                    m = g * 16 + j
                    a = (rows_v[r, pl.ds(0, 16)] * w1c[0]
                         + rows_v[r, pl.ds(16, 16)] * w1c[1]
                         + rows_v[r, pl.ds(32, 16)] * w1c[2]
                         + rows_v[r, pl.ds(48, 16)] * w1c[3])
                    for t in range(8):
                        a = a + mbuf[m, pl.ds(16 * t, 16)] * wmc[t]
                    ovec = jnp.where(lane == j, ovec + _tree_sum(a, lane),
                                     ovec)
                out_v[pl.ds(q * QR + g * 16, 16)] = ovec
                return carry

            lax.fori_loop(0, QR // 16, body, 0)
        pltpu.sync_copy(out_v, out_hbm.at[pl.ds(base, BPW)])

    return fused_k


_fused_cache = []


def kernel(user_ids, movie_features, user_embedding, W_feat, b_feat, W_fc, b_fc):
    if not _fused_cache:
        _fused_cache.append(_make_fused())
    params = jnp.concatenate(
        [W_fc[:, 0], b_feat, jnp.broadcast_to(b_fc, (16,))])
    return _fused_cache[0](user_embedding, user_ids.astype(jnp.int32),
                           movie_features, W_feat, params)


# speedup vs baseline: 5.5945x; 3.2432x over previous
import functools

import jax
import jax.numpy as jnp
from jax import lax
from jax.experimental import pallas as pl
from jax.experimental.pallas import tpu as pltpu
from jax.experimental.pallas import tpu_sc as plsc

B = 16384
D = 64
NF = 128
NU = 1000000
NC, NS = 2, 16
NW = NC * NS
BPW = B // NW
K = 64
NP = 208

UBLK = 32768
UGRID = -(-NU // UBLK)

_DNUMS = lax.GatherDimensionNumbers(
    offset_dims=(), collapsed_slice_dims=(0,), start_index_map=(0,))


def _shuf(v, idx):
    return lax.gather(v, idx[:, None], _DNUMS, (1,),
                      mode=lax.GatherScatterMode.PROMISE_IN_BOUNDS)


def _tree_sum(v, lane):
    for d in (1, 2, 4, 8):
        v = v + _shuf(v, lane ^ d)
    return v


def _udot_body(w1_ref, t_ref, out_ref):
    out_ref[...] = jnp.dot(w1_ref[...], t_ref[...],
                           preferred_element_type=jnp.float32).reshape(UBLK)


_udot = pl.pallas_call(
    _udot_body,
    grid=(UGRID,),
    in_specs=[
        pl.BlockSpec((1, D), lambda i: (0, 0)),
        pl.BlockSpec((D, UBLK), lambda i: (0, i)),
    ],
    out_specs=pl.BlockSpec((UBLK,), lambda i: (i,)),
    out_shape=jax.ShapeDtypeStruct((NU,), jnp.float32),
)


def _make_sc():
    mesh = plsc.VectorSubcoreMesh(core_axis_name="c", subcore_axis_name="s")

    @functools.partial(
        pl.kernel,
        mesh=mesh,
        out_type=jax.ShapeDtypeStruct((B,), jnp.float32),
        scratch_types=[
            pltpu.VMEM((BPW,), jnp.int32),
            pltpu.VMEM((BPW * 16,), jnp.float32),
            pltpu.VMEM((BPW // 4, NF), jnp.float32),
            pltpu.VMEM((BPW // 4, NF), jnp.float32),
            pltpu.VMEM((D, NF), jnp.float32),
            pltpu.VMEM((NP,), jnp.float32),
            pltpu.VMEM((NF,), jnp.float32),
            pltpu.VMEM((BPW,), jnp.float32),
            pltpu.SemaphoreType.DMA,
            pltpu.SemaphoreType.DMA,
            pltpu.SemaphoreType.DMA,
        ],
    )
    def sc_k(udot_hbm, idx_hbm, mf_hbm, wfeatT_hbm, params_hbm, out_hbm,
             idx_v, ud_v, mf_v0, mf_v1, wfeatT_v, par_v, wm_v, out_v,
             sem_g, sem_m0, sem_m1):
        wid = lax.axis_index("s") * NC + lax.axis_index("c")
        base = wid * BPW
        lane = lax.iota(jnp.int32, 16)
        QR = BPW // 4
        mf_bufs = (mf_v0, mf_v1)
        mf_sems = (sem_m0, sem_m1)

        pltpu.sync_copy(idx_hbm.at[pl.ds(base, BPW)], idx_v)
        pltpu.async_copy(mf_hbm.at[pl.ds(base, QR)], mf_v0, sem_m0)
        pltpu.sync_copy(wfeatT_hbm, wfeatT_v)
        pltpu.sync_copy(params_hbm, par_v)

        KG = K // 16

        def fire(g, carry):
            vec = idx_v[pl.ds(g * 16, 16)]
            for j in range(16):
                slab = (vec[j] // 16) * 16
                pltpu.async_copy(udot_hbm.at[pl.ds(slab, 16)],
                                 ud_v.at[pl.ds((g * 16 + j) * 16, 16)], sem_g)

            @pl.when(g >= KG)
            def _():
                pltpu.make_async_copy(
                    udot_hbm.at[pl.ds(0, 256)],
                    ud_v.at[pl.ds((g - KG) * 256, 256)],
                    sem_g).wait()

            return carry

        lax.fori_loop(0, BPW // 16, fire, 0)

        w2c = [par_v[pl.ds(D + 16 * t, 16)] for t in range(4)]

        def wm_body(g, carry):
            acc = jnp.zeros((16,), jnp.float32)
            for t in range(4):
                wv = w2c[t]
                for j in range(16):
                    acc = acc + wfeatT_v[16 * t + j, pl.ds(g * 16, 16)] * wv[j]
            wm_v[pl.ds(g * 16, 16)] = acc
            return carry

        lax.fori_loop(0, NF // 16, wm_body, 0)

        ca = (par_v[pl.ds(2 * D, 16)] * w2c[0]
              + par_v[pl.ds(2 * D + 16, 16)] * w2c[1]
              + par_v[pl.ds(2 * D + 32, 16)] * w2c[2]
              + par_v[pl.ds(2 * D + 48, 16)] * w2c[3])
        cvec = _tree_sum(ca, lane) + par_v[pl.ds(3 * D, 16)]

        def drain(g, carry):
            pltpu.make_async_copy(udot_hbm.at[pl.ds(0, 256)],
                                  ud_v.at[pl.ds((BPW - K + g * 16) * 16, 256)],
                                  sem_g).wait()
            return carry

        lax.fori_loop(0, KG, drain, 0)

        wmc = [wm_v[pl.ds(16 * t, 16)] for t in range(8)]

        for q in range(4):
            mbuf = mf_bufs[q % 2]
            pltpu.make_async_copy(mf_hbm.at[pl.ds(base + q * QR, QR)],
                                  mbuf, mf_sems[q % 2]).wait()
            if q + 1 < 4:
                pltpu.async_copy(mf_hbm.at[pl.ds(base + (q + 1) * QR, QR)],
                                 mf_bufs[(q + 1) % 2], mf_sems[(q + 1) % 2])

            def body(g, carry, q=q, mbuf=mbuf):
                u0 = q * QR + g * 16
                vec = idx_v[pl.ds(u0, 16)]
                acc = cvec
                for j in range(16):
                    m = g * 16 + j
                    uslab = ud_v[pl.ds((u0 + j) * 16, 16)]
                    uval = _shuf(uslab, jnp.full((16,), vec[j] % 16,
                                                 jnp.int32))
                    a = mbuf[m, pl.ds(0, 16)] * wmc[0]
                    for t in range(1, 8):
                        a = a + mbuf[m, pl.ds(16 * t, 16)] * wmc[t]
                    acc = acc + jnp.where(lane == j, uval + _tree_sum(a, lane),
                                          0.0)
                out_v[pl.ds(u0, 16)] = acc
                return carry

            lax.fori_loop(0, QR // 16, body, 0)
        pltpu.sync_copy(out_v, out_hbm.at[pl.ds(base, BPW)])

    return sc_k


_sc_cache = []


def kernel(user_ids, movie_features, user_embedding, W_feat, b_feat, W_fc, b_fc):
    if not _sc_cache:
        _sc_cache.append(_make_sc())
    w1 = W_fc[:D, 0].reshape(1, D)
    params = jnp.concatenate(
        [W_fc[:, 0], b_feat, jnp.broadcast_to(b_fc, (16,))])
    udot = _udot(w1, user_embedding.T)
    return _sc_cache[0](udot, user_ids.astype(jnp.int32),
                        movie_features, W_feat.T, params)


# speedup vs baseline: 5.9491x; 1.0634x over previous
import functools

import jax
import jax.numpy as jnp
from jax import lax
from jax.experimental import pallas as pl
from jax.experimental.pallas import tpu as pltpu
from jax.experimental.pallas import tpu_sc as plsc

B = 16384
D = 64
NF = 128
NU = 1000000
NC, NS = 2, 16
NW = NC * NS
BPW = B // NW
NP = 208

UBLK = 32768
UGRID = -(-NU // UBLK)

_DNUMS = lax.GatherDimensionNumbers(
    offset_dims=(), collapsed_slice_dims=(0,), start_index_map=(0,))


def _shuf(v, idx):
    return lax.gather(v, idx[:, None], _DNUMS, (1,),
                      mode=lax.GatherScatterMode.PROMISE_IN_BOUNDS)


def _tree_sum(v, lane):
    for d in (1, 2, 4, 8):
        v = v + _shuf(v, lane ^ d)
    return v


def _udot_body(w1_ref, t_ref, out_ref):
    out_ref[...] = jnp.dot(w1_ref[...], t_ref[...],
                           preferred_element_type=jnp.float32).reshape(UBLK)


_udot = pl.pallas_call(
    _udot_body,
    grid=(UGRID,),
    in_specs=[
        pl.BlockSpec((1, D), lambda i: (0, 0)),
        pl.BlockSpec((D, UBLK), lambda i: (0, i)),
    ],
    out_specs=pl.BlockSpec((UBLK,), lambda i: (i,)),
    out_shape=jax.ShapeDtypeStruct((NU,), jnp.float32),
)


def _make_sc():
    mesh = plsc.VectorSubcoreMesh(core_axis_name="c", subcore_axis_name="s")

    @functools.partial(
        pl.kernel,
        mesh=mesh,
        compiler_params=pltpu.CompilerParams(use_tc_tiling_on_sc=False),
        out_type=jax.ShapeDtypeStruct((B,), jnp.float32),
        scratch_types=[
            pltpu.VMEM((BPW,), jnp.int32),
            pltpu.VMEM((BPW,), jnp.float32),
            pltpu.VMEM((BPW // 4, NF), jnp.float32),
            pltpu.VMEM((BPW // 4, NF), jnp.float32),
            pltpu.VMEM((D, NF), jnp.float32),
            pltpu.VMEM((NP,), jnp.float32),
            pltpu.VMEM((NF,), jnp.float32),
            pltpu.VMEM((BPW,), jnp.float32),
            pltpu.SemaphoreType.DMA,
            pltpu.SemaphoreType.DMA,
            pltpu.SemaphoreType.DMA,
        ],
    )
    def sc_k(udot_hbm, idx_hbm, mf_hbm, wfeatT_hbm, params_hbm, out_hbm,
             idx_v, ud_v, mf_v0, mf_v1, wfeatT_v, par_v, wm_v, out_v,
             sem_g, sem_m0, sem_m1):
        wid = lax.axis_index("s") * NC + lax.axis_index("c")
        base = wid * BPW
        lane = lax.iota(jnp.int32, 16)
        QR = BPW // 4
        mf_bufs = (mf_v0, mf_v1)
        mf_sems = (sem_m0, sem_m1)

        pltpu.sync_copy(idx_hbm.at[pl.ds(base, BPW)], idx_v)
        pltpu.async_copy(mf_hbm.at[pl.ds(base, QR)], mf_v0, sem_m0)
        gcp = pltpu.async_copy(udot_hbm.at[idx_v], ud_v, sem_g)
        pltpu.sync_copy(wfeatT_hbm, wfeatT_v)
        pltpu.sync_copy(params_hbm, par_v)

        w2c = [par_v[pl.ds(D + 16 * t, 16)] for t in range(4)]

        def wm_body(g, carry):
            acc = jnp.zeros((16,), jnp.float32)
            for t in range(4):
                wv = w2c[t]
                for j in range(16):
                    acc = acc + wfeatT_v[16 * t + j, pl.ds(g * 16, 16)] * wv[j]
            wm_v[pl.ds(g * 16, 16)] = acc
            return carry

        lax.fori_loop(0, NF // 16, wm_body, 0)

        ca = (par_v[pl.ds(2 * D, 16)] * w2c[0]
              + par_v[pl.ds(2 * D + 16, 16)] * w2c[1]
              + par_v[pl.ds(2 * D + 32, 16)] * w2c[2]
              + par_v[pl.ds(2 * D + 48, 16)] * w2c[3])
        cvec = _tree_sum(ca, lane) + par_v[pl.ds(3 * D, 16)]

        gcp.wait()
        wmc = [wm_v[pl.ds(16 * t, 16)] for t in range(8)]

        for q in range(4):
            mbuf = mf_bufs[q % 2]
            pltpu.make_async_copy(mf_hbm.at[pl.ds(base + q * QR, QR)],
                                  mbuf, mf_sems[q % 2]).wait()
            if q + 1 < 4:
                pltpu.async_copy(mf_hbm.at[pl.ds(base + (q + 1) * QR, QR)],
                                 mf_bufs[(q + 1) % 2], mf_sems[(q + 1) % 2])

            def body(g, carry, q=q, mbuf=mbuf):
                u0 = q * QR + g * 16
                acc = cvec + ud_v[pl.ds(u0, 16)]
                for j in range(16):
                    m = g * 16 + j
                    a = mbuf[m, pl.ds(0, 16)] * wmc[0]
                    for t in range(1, 8):
                        a = a + mbuf[m, pl.ds(16 * t, 16)] * wmc[t]
                    acc = acc + jnp.where(lane == j, _tree_sum(a, lane), 0.0)
                out_v[pl.ds(u0, 16)] = acc
                return carry

            lax.fori_loop(0, QR // 16, body, 0)
        pltpu.sync_copy(out_v, out_hbm.at[pl.ds(base, BPW)])

    return sc_k


_sc_cache = []


def kernel(user_ids, movie_features, user_embedding, W_feat, b_feat, W_fc, b_fc):
    if not _sc_cache:
        _sc_cache.append(_make_sc())
    w1 = W_fc[:D, 0].reshape(1, D)
    params = jnp.concatenate(
        [W_fc[:, 0], b_feat, jnp.broadcast_to(b_fc, (16,))])
    udot = _udot(w1, user_embedding.T)
    return _sc_cache[0](udot, user_ids.astype(jnp.int32),
                        movie_features, W_feat.T, params)


# speedup vs baseline: 6.3744x; 1.0715x over previous
import functools

import jax
import jax.numpy as jnp
from jax import lax
from jax.experimental import pallas as pl
from jax.experimental.pallas import tpu as pltpu
from jax.experimental.pallas import tpu_sc as plsc

B = 16384
D = 64
NF = 128
NU = 1000000
NC, NS = 2, 16
NW = NC * NS
BPW = B // NW

UBLK = 32768
UGRID = -(-NU // UBLK)
NMB = 16
MBLK = B // NMB


def _tc_body(w1_ref, tT_ref, mf_ref, wfT_ref, w2_ref, bf_ref, bfc_ref,
             ud_ref, qm_ref):
    ud_ref[...] = jnp.dot(w1_ref[...], tT_ref[...],
                          preferred_element_type=jnp.float32).reshape(UBLK)
    wm_row = jnp.dot(w2_ref[...], wfT_ref[...],
                     preferred_element_type=jnp.float32)
    qm = lax.dot_general(mf_ref[...], wm_row, (((1,), (1,)), ((), ())),
                         preferred_element_type=jnp.float32)
    c = jnp.sum(bf_ref[...] * w2_ref[...]) + bfc_ref[0, 0]
    qm_ref[...] = qm[:, 0] + c


_tc_dense = pl.pallas_call(
    _tc_body,
    grid=(UGRID,),
    in_specs=[
        pl.BlockSpec((1, D), lambda i: (0, 0)),
        pl.BlockSpec((D, UBLK), lambda i: (0, i)),
        pl.BlockSpec((MBLK, NF), lambda i: (jnp.minimum(i, NMB - 1), 0)),
        pl.BlockSpec((D, NF), lambda i: (0, 0)),
        pl.BlockSpec((1, D), lambda i: (0, 0)),
        pl.BlockSpec((1, D), lambda i: (0, 0)),
        pl.BlockSpec((1, 1), lambda i: (0, 0)),
    ],
    out_specs=[
        pl.BlockSpec((UBLK,), lambda i: (i,)),
        pl.BlockSpec((MBLK,), lambda i: (jnp.minimum(i, NMB - 1),)),
    ],
    out_shape=[
        jax.ShapeDtypeStruct((NU,), jnp.float32),
        jax.ShapeDtypeStruct((B,), jnp.float32),
    ],
)


def _make_sc():
    mesh = plsc.VectorSubcoreMesh(core_axis_name="c", subcore_axis_name="s")

    @functools.partial(
        pl.kernel,
        mesh=mesh,
        compiler_params=pltpu.CompilerParams(use_tc_tiling_on_sc=False),
        out_type=jax.ShapeDtypeStruct((B,), jnp.float32),
        scratch_types=[
            pltpu.VMEM((BPW,), jnp.int32),
            pltpu.VMEM((BPW,), jnp.float32),
            pltpu.VMEM((BPW,), jnp.float32),
            pltpu.VMEM((BPW,), jnp.float32),
            pltpu.SemaphoreType.DMA,
        ],
    )
    def sc_k(udot_hbm, qm_hbm, idx_hbm, out_hbm,
             idx_v, ud_v, qm_v, out_v, sem_g):
        wid = lax.axis_index("s") * NC + lax.axis_index("c")
        base = wid * BPW

        pltpu.sync_copy(idx_hbm.at[pl.ds(base, BPW)], idx_v)
        gcp = pltpu.async_copy(udot_hbm.at[idx_v], ud_v, sem_g)
        pltpu.sync_copy(qm_hbm.at[pl.ds(base, BPW)], qm_v)
        gcp.wait()

        def body(g, carry):
            out_v[pl.ds(g * 16, 16)] = (ud_v[pl.ds(g * 16, 16)]
                                        + qm_v[pl.ds(g * 16, 16)])
            return carry

        lax.fori_loop(0, BPW // 16, body, 0)
        pltpu.sync_copy(out_v, out_hbm.at[pl.ds(base, BPW)])

    return sc_k


_sc_cache = []


def kernel(user_ids, movie_features, user_embedding, W_feat, b_feat, W_fc, b_fc):
    if not _sc_cache:
        _sc_cache.append(_make_sc())
    w1 = W_fc[:D, 0].reshape(1, D)
    w2 = W_fc[D:, 0].reshape(1, D)
    udot, qmovie = _tc_dense(w1, user_embedding.T, movie_features, W_feat.T,
                             w2, b_feat.reshape(1, D), b_fc.reshape(1, 1))
    return _sc_cache[0](udot, qmovie, user_ids.astype(jnp.int32))
